# baseline (speedup 1.0000x reference)
import jax
import jax.numpy as jnp
from jax import lax
from jax.experimental import pallas as pl
from jax.experimental.pallas import tpu as pltpu
from jax.experimental.pallas import tpu_sc as plsc

_L = 200
_B = 1024
_M = 64
_N = _L * _B
_NG = _N - _B
_SUB = 128
_UNIT = 512
_NSUB = _UNIT // _SUB
_NUNITS = _NG // _UNIT
_NTASK = 3
_TOT = _NTASK * _NUNITS
_NW = 32
_ZROWS = _B // _NW
_RYOFF = _B // _SUB


def _body(ly_h, lp_h, ry_h, emb_h, pos_h, lo_h, po_h, ro_h,
          idx_v, rows_v, sem_g, sem_s):
    c = lax.axis_index("c")
    s = lax.axis_index("s")
    w = s * 2 + c

    zvec = jnp.zeros((16,), jnp.float32)

    def _zrow(r, carry):
        for cc in range(_M // 16):
            rows_v[0, r, pl.ds(cc * 16, 16)] = zvec
        return carry

    lax.fori_loop(0, _ZROWS, _zrow, 0)
    zbase = w * _ZROWS
    for out_h in (lo_h, po_h, ro_h):
        pltpu.sync_copy(rows_v.at[0, pl.ds(0, _ZROWS)],
                        out_h.at[pl.ds(zbase, _ZROWS)])

    nu = (_TOT // _NW) + jnp.where(w < (_TOT % _NW), 1, 0)

    def _split(uid):
        task = uid % _NTASK
        u = uid // _NTASK
        return task, u, _B + u * _UNIT, u * _NSUB

    def _fire_gathers(uid, b):
        task, _, _, irow = _split(uid)

        def _one(idx_h, tab_h, ioff):
            pltpu.sync_copy(idx_h.at[pl.ds(ioff, _NSUB)], idx_v.at[b])
            for j in range(_NSUB):
                pltpu.async_copy(tab_h.at[idx_v.at[b, j]],
                                 rows_v.at[b, pl.ds(j * _SUB, _SUB)], sem_g)

        @pl.when(task == 0)
        def _():
            _one(ly_h, emb_h, irow)

        @pl.when(task == 1)
        def _():
            _one(lp_h, pos_h, irow)

        @pl.when(task == 2)
        def _():
            _one(ry_h, emb_h, _RYOFF + irow)

    def _wait_gathers(uid, b):
        task, _, _, irow = _split(uid)

        def _one(tab_h):
            for j in range(_NSUB):
                pltpu.make_async_copy(
                    tab_h.at[idx_v.at[b, j]],
                    rows_v.at[b, pl.ds(j * _SUB, _SUB)], sem_g).wait()

        @pl.when(task == 1)
        def _():
            _one(pos_h)

        @pl.when(task != 1)
        def _():
            _one(emb_h)

    def _store(uid, b):
        task, _, orow, _ = _split(uid)

        @pl.when(task == 0)
        def _():
            pltpu.async_copy(rows_v.at[b], lo_h.at[pl.ds(orow, _UNIT)], sem_s)

        @pl.when(task == 1)
        def _():
            pltpu.async_copy(rows_v.at[b], po_h.at[pl.ds(orow, _UNIT)], sem_s)

        @pl.when(task == 2)
        def _():
            pltpu.async_copy(rows_v.at[b], ro_h.at[pl.ds(orow, _UNIT)], sem_s)

    def _wait_store():
        pltpu.make_async_copy(emb_h.at[pl.ds(0, _UNIT)], rows_v.at[0],
                              sem_s).wait()

    _fire_gathers(w, 0)

    def _step(i, carry):
        b = i % 2
        uid = w + i * _NW
        _wait_gathers(uid, b)
        _store(uid, b)

        @pl.when(i + 1 < nu)
        def _():
            _fire_gathers(uid + _NW, 1 - b)

        _wait_store()
        return carry

    lax.fori_loop(0, nu, _step, 0)


@jax.jit
def kernel(ly, lp, ry, emb_table, pos_table):
    ly2 = ly.astype(jnp.int32).reshape(_N // _SUB, _SUB)
    lp2 = lp.astype(jnp.int32).reshape(_N // _SUB, _SUB)
    ry2 = ry.astype(jnp.int32).reshape(_N // _SUB, _SUB)

    mesh = plsc.VectorSubcoreMesh(core_axis_name="c", subcore_axis_name="s")
    out3 = (jax.ShapeDtypeStruct((_N, _M), jnp.float32),) * 3
    run = pl.kernel(
        _body,
        mesh=mesh,
        out_type=out3,
        scratch_types=[
            pltpu.VMEM((2, _NSUB, _SUB), jnp.int32),
            pltpu.VMEM((2, _UNIT, _M), jnp.float32),
            pltpu.SemaphoreType.DMA,
            pltpu.SemaphoreType.DMA,
        ],
        compiler_params=pltpu.CompilerParams(use_tc_tiling_on_sc=False),
    )
    lo, po, ro = run(ly2, lp2, ry2, emb_table, pos_table)
    return (lo.reshape(_L, _B, _M),
            po.reshape(_L, _B, _M),
            ro.reshape(_L, _B, _M))
